# baseline (speedup 1.0000x reference)
import functools

import jax
import jax.numpy as jnp
from jax import lax
from jax.experimental import pallas as pl
from jax.experimental.pallas import tpu as pltpu
from jax.experimental.pallas import tpu_sc as plsc

NC, NS = 2, 16
CHUNK = 128
SPLIT = 4
SUB = CHUNK // SPLIT
NBUF = 8
KFAST = 160
KSLOW = 0
FAST_C = 0
STAGE = 40
ROW_BLK = 400


def _sc_segment_sum(x, src_r, dst_r, n_pad, d):
    rows_per_sub = n_pad // NS
    mesh = plsc.VectorSubcoreMesh(core_axis_name="c", subcore_axis_name="s")

    @functools.partial(
        pl.kernel,
        out_type=jax.ShapeDtypeStruct((NC, n_pad, d), jnp.float32),
        mesh=mesh,
        scratch_types=(
            [pltpu.VMEM((STAGE, CHUNK), jnp.int32)] * 2
            + [pltpu.VMEM((SUB, d), jnp.float32)] * NBUF
            + [pltpu.VMEM_SHARED((n_pad, d), jnp.float32)]
            + [pltpu.SemaphoreType.DMA] * NBUF
        ),
    )
    def body(x_hbm, zeros_hbm, src_hbm, dst_hbm, out_hbm, *scr):
        src_v, dst_v = scr[0], scr[1]
        bufs = scr[2:2 + NBUF]
        agg_sh = scr[2 + NBUF]
        sems = scr[3 + NBUF:]
        c = lax.axis_index("c")
        s = lax.axis_index("s")
        base = s * rows_per_sub

        pltpu.sync_copy(zeros_hbm.at[pl.ds(base, rows_per_sub)],
                        agg_sh.at[pl.ds(base, rows_per_sub)])
        plsc.subcore_barrier()

        def edge_pass(row0, nch):
            if nch == 0:
                return
            ph = min(nch, STAGE)
            nsub = ph * SPLIT
            for phase in range(nch // ph):
                pltpu.sync_copy(src_hbm.at[pl.ds(row0 + phase * ph, ph)],
                                src_v.at[pl.ds(0, ph)])
                pltpu.sync_copy(dst_hbm.at[pl.ds(row0 + phase * ph, ph)],
                                dst_v.at[pl.ds(0, ph)])
                for b in range(NBUF):
                    pltpu.async_copy(
                        x_hbm.at[src_v.at[b // SPLIT, pl.ds((b % SPLIT) * SUB, SUB)]],
                        bufs[b], sems[b])

                def step(i, _):
                    for b in range(NBUF):
                        row = (NBUF // SPLIT) * i + b // SPLIT
                        col = (b % SPLIT) * SUB
                        pltpu.make_async_copy(
                            x_hbm.at[src_v.at[row, pl.ds(col, SUB)]],
                            bufs[b], sems[b]).wait()
                        pltpu.sync_copy(
                            bufs[b], agg_sh.at[dst_v.at[row, pl.ds(col, SUB)]],
                            add=True)

                        @pl.when(i < nsub // NBUF - 1)
                        def _():
                            nrow = (NBUF // SPLIT) * (i + 1) + b // SPLIT
                            pltpu.async_copy(
                                x_hbm.at[src_v.at[nrow, pl.ds(col, SUB)]],
                                bufs[b], sems[b])

                    return _

                lax.fori_loop(0, nsub // NBUF, step, None)

        @pl.when(c == FAST_C)
        def _():
            edge_pass(s * KFAST, KFAST)

        @pl.when(c != FAST_C)
        def _():
            edge_pass(NS * KFAST + s * KSLOW, KSLOW)

        plsc.subcore_barrier()

        pltpu.sync_copy(agg_sh.at[pl.ds(base, rows_per_sub)],
                        out_hbm.at[c, pl.ds(base, rows_per_sub)])

    zeros = jnp.zeros((n_pad, d), jnp.float32)
    return body(x, zeros, src_r, dst_r)


def _dot(a, b):
    return jnp.dot(a, b, preferred_element_type=jnp.float32,
                   precision=lax.Precision.HIGHEST)


def _mlp_body(p_ref, x_ref, w1_ref, b1_ref, w2_ref, b2_ref, w3_ref, b3_ref, o_ref):
    d = x_ref.shape[-1]
    agg = p_ref[0] + p_ref[1]
    xb = x_ref[...]
    h = _dot(agg, w1_ref[:d, :]) + _dot(xb, w1_ref[d:, :]) + b1_ref[...]
    h = jnp.maximum(h, 0.0)
    h = jnp.maximum(_dot(h, w2_ref[...]) + b2_ref[...], 0.0)
    o_ref[...] = _dot(h, w3_ref[...]) + b3_ref[...]


def _tc_mlp(partial, x, W1, b1, W2, b2, W3, b3):
    n, d = x.shape
    h1 = W1.shape[1]
    h2 = W2.shape[1]
    nblk = n // ROW_BLK
    return pl.pallas_call(
        _mlp_body,
        grid=(nblk,),
        in_specs=[
            pl.BlockSpec((NC, ROW_BLK, d), lambda i: (0, i, 0)),
            pl.BlockSpec((ROW_BLK, d), lambda i: (i, 0)),
            pl.BlockSpec((2 * d, h1), lambda i: (0, 0)),
            pl.BlockSpec((1, h1), lambda i: (0, 0)),
            pl.BlockSpec((h1, h2), lambda i: (0, 0)),
            pl.BlockSpec((1, h2), lambda i: (0, 0)),
            pl.BlockSpec((h2, d), lambda i: (0, 0)),
            pl.BlockSpec((1, d), lambda i: (0, 0)),
        ],
        out_specs=pl.BlockSpec((ROW_BLK, d), lambda i: (i, 0)),
        out_shape=jax.ShapeDtypeStruct((n, d), jnp.float32),
    )(partial, x, W1, b1.reshape(1, -1), W2, b2.reshape(1, -1),
      W3, b3.reshape(1, -1))


def kernel(node_feature, edge_index, W1, b1, W2, b2, W3, b3):
    n, d = node_feature.shape
    e = edge_index.shape[1]

    nrows = NS * (KFAST + KSLOW)
    e_pad = nrows * CHUNK
    n_pad = -(-(n + 8) // (NS * 8)) * (NS * 8)
    pad = e_pad - e
    src = edge_index[0]
    dst = edge_index[1]
    src_p = jnp.concatenate(
        [src, jnp.zeros((pad,), jnp.int32)]).reshape(nrows, CHUNK)
    dst_pad_vals = n + (jnp.arange(pad, dtype=jnp.int32) % (n_pad - n))
    dst_p = jnp.concatenate([dst, dst_pad_vals]).reshape(nrows, CHUNK)

    partial = _sc_segment_sum(node_feature, src_p, dst_p, n_pad, d)
    return _tc_mlp(partial, node_feature, W1, b1, W2, b2, W3, b3)


# speedup vs baseline: 1.2867x; 1.2867x over previous
import functools

import jax
import jax.numpy as jnp
from jax import lax
from jax.experimental import pallas as pl
from jax.experimental.pallas import tpu as pltpu
from jax.experimental.pallas import tpu_sc as plsc

NC, NS = 2, 16
CHUNK = 128
SPLIT = 4
SUB = CHUNK // SPLIT
NBUF = 8
KFAST = 152
KSLOW = 8
FAST_C = 0
STAGE = 40
ROW_BLK = 400


def _sc_segment_sum(x, src_r, dst_r, n_pad, d):
    rows_per_sub = n_pad // NS
    mesh = plsc.VectorSubcoreMesh(core_axis_name="c", subcore_axis_name="s")

    @functools.partial(
        pl.kernel,
        out_type=jax.ShapeDtypeStruct((NC, n_pad, d), jnp.float32),
        mesh=mesh,
        scratch_types=(
            [pltpu.VMEM((STAGE, CHUNK), jnp.int32)] * 2
            + [pltpu.VMEM((SUB, d), jnp.float32)] * NBUF
            + [pltpu.VMEM_SHARED((n_pad, d), jnp.float32)]
            + [pltpu.SemaphoreType.DMA] * NBUF
        ),
    )
    def body(x_hbm, zeros_hbm, src_hbm, dst_hbm, out_hbm, *scr):
        src_v, dst_v = scr[0], scr[1]
        bufs = scr[2:2 + NBUF]
        agg_sh = scr[2 + NBUF]
        sems = scr[3 + NBUF:]
        c = lax.axis_index("c")
        s = lax.axis_index("s")
        base = s * rows_per_sub

        pltpu.sync_copy(zeros_hbm.at[pl.ds(base, rows_per_sub)],
                        agg_sh.at[pl.ds(base, rows_per_sub)])
        plsc.subcore_barrier()

        def edge_pass(row0, nch):
            if nch == 0:
                return
            offs = []
            o = 0
            while o < nch:
                offs.append((o, min(STAGE, nch - o)))
                o += STAGE
            for off, ph in offs:
                nsub = ph * SPLIT
                pltpu.sync_copy(src_hbm.at[pl.ds(row0 + off, ph)],
                                src_v.at[pl.ds(0, ph)])
                pltpu.sync_copy(dst_hbm.at[pl.ds(row0 + off, ph)],
                                dst_v.at[pl.ds(0, ph)])
                for b in range(NBUF):
                    pltpu.async_copy(
                        x_hbm.at[src_v.at[b // SPLIT, pl.ds((b % SPLIT) * SUB, SUB)]],
                        bufs[b], sems[b])

                def step(i, _):
                    for b in range(NBUF):
                        row = (NBUF // SPLIT) * i + b // SPLIT
                        col = (b % SPLIT) * SUB
                        pltpu.make_async_copy(
                            x_hbm.at[src_v.at[row, pl.ds(col, SUB)]],
                            bufs[b], sems[b]).wait()
                        pltpu.sync_copy(
                            bufs[b], agg_sh.at[dst_v.at[row, pl.ds(col, SUB)]],
                            add=True)

                        @pl.when(i < nsub // NBUF - 1)
                        def _():
                            nrow = (NBUF // SPLIT) * (i + 1) + b // SPLIT
                            pltpu.async_copy(
                                x_hbm.at[src_v.at[nrow, pl.ds(col, SUB)]],
                                bufs[b], sems[b])

                    return _

                lax.fori_loop(0, nsub // NBUF, step, None)

        @pl.when(c == FAST_C)
        def _():
            edge_pass(s * KFAST, KFAST)

        @pl.when(c != FAST_C)
        def _():
            edge_pass(NS * KFAST + s * KSLOW, KSLOW)

        plsc.subcore_barrier()

        pltpu.sync_copy(agg_sh.at[pl.ds(base, rows_per_sub)],
                        out_hbm.at[c, pl.ds(base, rows_per_sub)])

    zeros = jnp.zeros((n_pad, d), jnp.float32)
    return body(x, zeros, src_r, dst_r)


def _dot(a, b):
    return jnp.dot(a, b, preferred_element_type=jnp.float32,
                   precision=lax.Precision.HIGHEST)


def _mlp_body(p_ref, x_ref, w1_ref, b1_ref, w2_ref, b2_ref, w3_ref, b3_ref, o_ref):
    d = x_ref.shape[-1]
    agg = p_ref[0] + p_ref[1]
    xb = x_ref[...]
    h = _dot(agg, w1_ref[:d, :]) + _dot(xb, w1_ref[d:, :]) + b1_ref[...]
    h = jnp.maximum(h, 0.0)
    h = jnp.maximum(_dot(h, w2_ref[...]) + b2_ref[...], 0.0)
    o_ref[...] = _dot(h, w3_ref[...]) + b3_ref[...]


def _tc_mlp(partial, x, W1, b1, W2, b2, W3, b3):
    n, d = x.shape
    h1 = W1.shape[1]
    h2 = W2.shape[1]
    nblk = n // ROW_BLK
    return pl.pallas_call(
        _mlp_body,
        grid=(nblk,),
        in_specs=[
            pl.BlockSpec((NC, ROW_BLK, d), lambda i: (0, i, 0)),
            pl.BlockSpec((ROW_BLK, d), lambda i: (i, 0)),
            pl.BlockSpec((2 * d, h1), lambda i: (0, 0)),
            pl.BlockSpec((1, h1), lambda i: (0, 0)),
            pl.BlockSpec((h1, h2), lambda i: (0, 0)),
            pl.BlockSpec((1, h2), lambda i: (0, 0)),
            pl.BlockSpec((h2, d), lambda i: (0, 0)),
            pl.BlockSpec((1, d), lambda i: (0, 0)),
        ],
        out_specs=pl.BlockSpec((ROW_BLK, d), lambda i: (i, 0)),
        out_shape=jax.ShapeDtypeStruct((n, d), jnp.float32),
    )(partial, x, W1, b1.reshape(1, -1), W2, b2.reshape(1, -1),
      W3, b3.reshape(1, -1))


def kernel(node_feature, edge_index, W1, b1, W2, b2, W3, b3):
    n, d = node_feature.shape
    e = edge_index.shape[1]

    nrows = NS * (KFAST + KSLOW)
    e_pad = nrows * CHUNK
    n_pad = -(-(n + 8) // (NS * 8)) * (NS * 8)
    pad = e_pad - e
    src = edge_index[0]
    dst = edge_index[1]
    src_p = jnp.concatenate(
        [src, jnp.zeros((pad,), jnp.int32)]).reshape(nrows, CHUNK)
    dst_pad_vals = n + (jnp.arange(pad, dtype=jnp.int32) % (n_pad - n))
    dst_p = jnp.concatenate([dst, dst_pad_vals]).reshape(nrows, CHUNK)

    partial = _sc_segment_sum(node_feature, src_p, dst_p, n_pad, d)
    return _tc_mlp(partial, node_feature, W1, b1, W2, b2, W3, b3)
